# baseline (speedup 1.0000x reference)
import functools

import jax
import jax.numpy as jnp
from jax import lax
from jax.experimental import pallas as pl
from jax.experimental.pallas import tpu as pltpu
from jax.experimental.pallas import tpu_sc as plsc

_N = 10000
_E = 320000
_DI = 128
_DE = 16
_H = 64
_L = 3

_NC = 2
_NS = 16
_NW = _NC * _NS
_C = 128
_CH = 79
_EW = _CH * _C
_EP = _NW * _EW
_NP = 10240
_RPT = _NP // _NS
_NB = 5
_CS = 64
_CHS = _EW // _CS
_NBS = 5

_BN = 2000
_BE = 4096


def _mm(a, b):
    return lax.dot_general(a, b, (((1,), (0,)), ((), ())),
                           preferred_element_type=jnp.float32)


def _sc_mesh():
    return plsc.VectorSubcoreMesh(core_axis_name="c", subcore_axis_name="s",
                                  num_cores=_NC, num_subcores=_NS)


def _replicated(shape):
    return pl.BlockSpec(shape, lambda i: (0,) * len(shape))


def _gather_rows(tbl, idx4):
    @functools.partial(
        pl.kernel,
        out_type=jax.ShapeDtypeStruct((_EP, 128), jnp.float32),
        mesh=_sc_mesh(),
        scratch_types=[
            pltpu.VMEM((_CH, _C), jnp.int32),
            [pltpu.VMEM((_C, 128), jnp.float32)] * _NB,
            [pltpu.SemaphoreType.DMA] * _NB,
            [pltpu.SemaphoreType.DMA] * _NB,
        ],
    )
    def k(tbl_hbm, idx_hbm, out_hbm, idx_v, bufs, gsems, wsems):
        c = lax.axis_index("c")
        s = lax.axis_index("s")
        base = (c * _NS + s) * _EW
        pltpu.sync_copy(idx_hbm.at[c, s], idx_v)
        pltpu.async_copy(tbl_hbm.at[idx_v.at[0]], bufs[0], gsems[0])
        pltpu.async_copy(tbl_hbm.at[idx_v.at[1]], bufs[1], gsems[1])

        def out_at(j):
            return out_hbm.at[pl.ds(base + j * _C, _C)]

        def body(i, carry):
            for b in range(_NB):
                j = i * _NB + b
                b2 = (b + 2) % _NB

                @pl.when(jnp.logical_and(j >= 3, j < _CH))
                def _():
                    pltpu.make_async_copy(bufs[b2], out_at(0), wsems[b2]).wait()

                @pl.when(j + 2 < _CH)
                def _():
                    pltpu.async_copy(tbl_hbm.at[idx_v.at[j + 2]], bufs[b2],
                                     gsems[b2])

                @pl.when(j < _CH)
                def _():
                    pltpu.make_async_copy(tbl_hbm.at[idx_v.at[j]], bufs[b],
                                          gsems[b]).wait()
                    pltpu.async_copy(bufs[b], out_at(j), wsems[b])
            return carry

        lax.fori_loop(0, (_CH + _NB - 1) // _NB, body, 0)
        for j in range(_CH - 3, _CH):
            b = j % _NB
            pltpu.make_async_copy(bufs[b], out_at(0), wsems[b]).wait()

    return k(tbl, idx4)


def _scatter_core(with_counts):
    out_types = [jax.ShapeDtypeStruct((_NC, _NP, _H), jnp.float32)]
    scratch = [
        pltpu.VMEM((_CHS, _CS), jnp.int32),
        [pltpu.VMEM((_CS, _H), jnp.float32)] * _NBS,
        [pltpu.SemaphoreType.DMA] * _NBS,
        [pltpu.SemaphoreType.DMA] * _NBS,
        pltpu.VMEM_SHARED((_NP, _H), jnp.float32),
    ]
    if with_counts:
        out_types.append(jax.ShapeDtypeStruct((_NC, _NP, 16), jnp.float32))
        scratch += [pltpu.VMEM((_CS, 16), jnp.float32),
                    pltpu.SemaphoreType.DMA,
                    pltpu.VMEM_SHARED((_NP, 16), jnp.float32)]

    def k(msg_hbm, idx_hbm, z_hbm, *rest):
        if with_counts:
            (zc_hbm, on_hbm, sum_hbm, cnt_hbm,
             idx_v, bufs, lsems, asems, acc, one_v, csem, cacc) = rest
        else:
            sum_hbm, idx_v, bufs, lsems, asems, acc = rest
        c = lax.axis_index("c")
        s = lax.axis_index("s")
        base = (c * _NS + s) * _EW
        rows = pl.ds(s * _RPT, _RPT)
        pltpu.sync_copy(idx_hbm.at[c, s], idx_v)
        pltpu.sync_copy(z_hbm, acc.at[rows])
        if with_counts:
            pltpu.sync_copy(on_hbm, one_v)
            pltpu.sync_copy(zc_hbm, cacc.at[rows])
        plsc.subcore_barrier()

        def msg_at(j):
            return msg_hbm.at[pl.ds(base + j * _CS, _CS)]

        pltpu.async_copy(msg_at(0), bufs[0], lsems[0])
        pltpu.async_copy(msg_at(1), bufs[1], lsems[1])

        def body(i, carry):
            for b in range(_NBS):
                j = i * _NBS + b
                b2 = (b + 2) % _NBS

                @pl.when(jnp.logical_and(j >= _NBS - 2, j < _CHS))
                def _():
                    pltpu.make_async_copy(bufs[b2], acc.at[idx_v.at[0]],
                                          asems[b2]).wait()

                @pl.when(j + 2 < _CHS)
                def _():
                    pltpu.async_copy(msg_at(j + 2), bufs[b2], lsems[b2])

                @pl.when(j < _CHS)
                def _():
                    pltpu.make_async_copy(msg_at(j), bufs[b], lsems[b]).wait()
                    pltpu.async_copy(bufs[b], acc.at[idx_v.at[j]], asems[b],
                                     add=True)
                    if with_counts:
                        @pl.when(j >= 1)
                        def _():
                            pltpu.make_async_copy(one_v, cacc.at[idx_v.at[0]],
                                                  csem).wait()
                        pltpu.async_copy(one_v, cacc.at[idx_v.at[j]], csem,
                                         add=True)
            return carry

        lax.fori_loop(0, (_CHS + _NBS - 1) // _NBS, body, 0)
        for j in range(_CHS - (_NBS - 2), _CHS):
            b = j % _NBS
            pltpu.make_async_copy(bufs[b], acc.at[idx_v.at[0]], asems[b]).wait()
        if with_counts:
            pltpu.make_async_copy(one_v, cacc.at[idx_v.at[0]], csem).wait()
        plsc.subcore_barrier()
        pltpu.sync_copy(acc.at[rows], sum_hbm.at[c, rows])
        if with_counts:
            pltpu.sync_copy(cacc.at[rows], cnt_hbm.at[c, rows])

    return functools.partial(
        pl.kernel, k,
        out_type=out_types if with_counts else out_types[0],
        mesh=_sc_mesh(),
        compiler_params=pltpu.CompilerParams(use_tc_tiling_on_sc=False),
        scratch_types=scratch,
    )()


def _scatter_add(msgs, idx4, z_h):
    return _scatter_core(False)(msgs, idx4, z_h)


def _scatter_add_cnt(msgs, idx4, z_h, z_c, o_c):
    return _scatter_core(True)(msgs, idx4, z_h, z_c, o_c)


def _encode(x, w1, b1, w2, b2, mt, mb):
    def body(x_r, w1_r, b1_r, w2_r, b2_r, mt_r, mb_r, h_r, t_r):
        h = jnp.maximum(_mm(x_r[...], w1_r[...]) + b1_r[...], 0.0)
        h = jnp.maximum(_mm(h, w2_r[...]) + b2_r[...], 0.0)
        h_r[...] = h
        t = _mm(h, mt_r[...]) + mb_r[...]
        t_r[...] = jnp.concatenate([t, jnp.zeros((_BN, _H), jnp.float32)],
                                   axis=-1)

    return pl.pallas_call(
        body,
        grid=(_N // _BN,),
        in_specs=[
            pl.BlockSpec((_BN, _DI), lambda i: (i, 0)),
            _replicated((_DI, _H)), _replicated((1, _H)),
            _replicated((_H, _H)), _replicated((1, _H)),
            _replicated((_H, _H)), _replicated((1, _H)),
        ],
        out_specs=[pl.BlockSpec((_BN, _H), lambda i: (i, 0)),
                   pl.BlockSpec((_BN, 128), lambda i: (i, 0))],
        out_shape=[jax.ShapeDtypeStruct((_N, _H), jnp.float32),
                   jax.ShapeDtypeStruct((_N, 128), jnp.float32)],
    )(x, w1, b1, w2, b2, mt, mb)


def _messages(g, ef, eew, eeb, w1b, w2, b2):
    def body(g_r, ef_r, eew_r, eeb_r, w1b_r, w2_r, b2_r, o_r):
        eh = jnp.maximum(_mm(ef_r[...], eew_r[...]) + eeb_r[...], 0.0)
        hid = jnp.maximum(g_r[:, :_H] + _mm(eh, w1b_r[...]), 0.0)
        o_r[...] = jnp.maximum(_mm(hid, w2_r[...]) + b2_r[...], 0.0)

    return pl.pallas_call(
        body,
        grid=(_EP // _BE,),
        in_specs=[
            pl.BlockSpec((_BE, 128), lambda i: (i, 0)),
            pl.BlockSpec((_BE, _DE), lambda i: (i, 0)),
            _replicated((_DE, _H)), _replicated((1, _H)),
            _replicated((_H, _H)), _replicated((_H, _H)), _replicated((1, _H)),
        ],
        out_specs=pl.BlockSpec((_BE, _H), lambda i: (i, 0)),
        out_shape=jax.ShapeDtypeStruct((_EP, _H), jnp.float32),
    )(g, ef, eew, eeb, w1b, w2, b2)


def _update(h, p0, p1, c0, c1, uw1t, uw1b, ub1, uw2, ub2, mt, mb):
    def body(h_r, p0_r, p1_r, c0_r, c1_r, uw1t_r, uw1b_r, ub1_r, uw2_r, ub2_r,
             mt_r, mb_r, hn_r, tn_r):
        cnt = c0_r[...] + c1_r[...]
        agg = (p0_r[...] + p1_r[...]) / jnp.maximum(cnt[:, :1], 1.0)
        hid = jnp.maximum(_mm(h_r[...], uw1t_r[...]) + _mm(agg, uw1b_r[...])
                          + ub1_r[...], 0.0)
        hn = jnp.maximum(_mm(hid, uw2_r[...]) + ub2_r[...], 0.0)
        hn_r[...] = hn
        t = _mm(hn, mt_r[...]) + mb_r[...]
        tn_r[...] = jnp.concatenate([t, jnp.zeros((_BN, _H), jnp.float32)],
                                    axis=-1)

    return pl.pallas_call(
        body,
        grid=(_N // _BN,),
        in_specs=[
            pl.BlockSpec((_BN, _H), lambda i: (i, 0)),
            pl.BlockSpec((_BN, _H), lambda i: (i, 0)),
            pl.BlockSpec((_BN, _H), lambda i: (i, 0)),
            pl.BlockSpec((_BN, 16), lambda i: (i, 0)),
            pl.BlockSpec((_BN, 16), lambda i: (i, 0)),
            _replicated((_H, _H)), _replicated((_H, _H)), _replicated((1, _H)),
            _replicated((_H, _H)), _replicated((1, _H)),
            _replicated((_H, _H)), _replicated((1, _H)),
        ],
        out_specs=[pl.BlockSpec((_BN, _H), lambda i: (i, 0)),
                   pl.BlockSpec((_BN, 128), lambda i: (i, 0))],
        out_shape=[jax.ShapeDtypeStruct((_N, _H), jnp.float32),
                   jax.ShapeDtypeStruct((_N, 128), jnp.float32)],
    )(h, p0, p1, c0, c1, uw1t, uw1b, ub1, uw2, ub2, mt, mb)


def _final(h, p0, p1, c0, c1, uw1t, uw1b, ub1, uw2, ub2, ow1, ob1, ow2p, ob2p):
    def body(h_r, p0_r, p1_r, c0_r, c1_r, uw1t_r, uw1b_r, ub1_r, uw2_r, ub2_r,
             ow1_r, ob1_r, ow2_r, ob2_r, o_r):
        cnt = c0_r[...] + c1_r[...]
        agg = (p0_r[...] + p1_r[...]) / jnp.maximum(cnt[:, :1], 1.0)
        hid = jnp.maximum(_mm(h_r[...], uw1t_r[...]) + _mm(agg, uw1b_r[...])
                          + ub1_r[...], 0.0)
        hn = jnp.maximum(_mm(hid, uw2_r[...]) + ub2_r[...], 0.0)
        hid2 = jnp.maximum(_mm(hn, ow1_r[...]) + ob1_r[...], 0.0)
        pred = _mm(hid2, ow2_r[...]) + ob2_r[...]
        o_r[...] = (2.0 * jnp.pi) / (1.0 + jnp.exp(-pred))

    return pl.pallas_call(
        body,
        grid=(_N // _BN,),
        in_specs=[
            pl.BlockSpec((_BN, _H), lambda i: (i, 0)),
            pl.BlockSpec((_BN, _H), lambda i: (i, 0)),
            pl.BlockSpec((_BN, _H), lambda i: (i, 0)),
            pl.BlockSpec((_BN, 16), lambda i: (i, 0)),
            pl.BlockSpec((_BN, 16), lambda i: (i, 0)),
            _replicated((_H, _H)), _replicated((_H, _H)), _replicated((1, _H)),
            _replicated((_H, _H)), _replicated((1, _H)),
            _replicated((_H, _H)), _replicated((1, _H)),
            _replicated((_H, 128)), _replicated((1, 128)),
        ],
        out_specs=pl.BlockSpec((_BN, 128), lambda i: (i, 0)),
        out_shape=jax.ShapeDtypeStruct((_N, 128), jnp.float32),
    )(h, p0, p1, c0, c1, uw1t, uw1b, ub1, uw2, ub2, ow1, ob1, ow2p, ob2p)


def kernel(node_features, edge_index, edge_features, ne_w1, ne_b1, ne_w2, ne_b2,
           ee_w1, ee_b1, msg_w1, msg_b1, msg_w2, msg_b2,
           upd_w1, upd_b1, upd_w2, upd_b2, out_w1, out_b1, out_w2, out_b2):
    row = lambda v: v.reshape(1, -1)
    out_dim = out_w2.shape[1]
    npad = _EP - _E
    src4 = jnp.pad(edge_index[0], (0, npad)).reshape(_NC, _NS, _CH, _C)
    dst4 = jnp.pad(edge_index[1], (0, npad),
                   constant_values=_NP - 1).reshape(_NC, _NS, _CHS, _CS)
    z_h = jnp.zeros((_RPT, _H), jnp.float32)
    z_c = jnp.zeros((_RPT, 16), jnp.float32)
    o_c = jnp.ones((_CS, 16), jnp.float32)
    ow2p = jnp.pad(out_w2, ((0, 0), (0, 128 - out_dim)))
    ob2p = row(jnp.pad(out_b2, (0, 128 - out_dim)))

    h, t = _encode(node_features, ne_w1, row(ne_b1), ne_w2, row(ne_b2),
                   msg_w1[0, :_H], row(msg_b1[0]))
    c0 = c1 = None
    for l in range(_L):
        g = _gather_rows(t, src4)
        msgs = _messages(g, edge_features, ee_w1, row(ee_b1),
                         msg_w1[l, _H:], msg_w2[l], row(msg_b2[l]))
        if l == 0:
            sums, cnts = _scatter_add_cnt(msgs, dst4, z_h, z_c, o_c)
            c0, c1 = cnts[0], cnts[1]
        else:
            sums = _scatter_add(msgs, dst4, z_h)
        if l < _L - 1:
            h, t = _update(h, sums[0], sums[1], c0, c1,
                           upd_w1[l, :_H], upd_w1[l, _H:], row(upd_b1[l]),
                           upd_w2[l], row(upd_b2[l]),
                           msg_w1[l + 1, :_H], row(msg_b1[l + 1]))
        else:
            pred = _final(h, sums[0], sums[1], c0, c1,
                          upd_w1[l, :_H], upd_w1[l, _H:], row(upd_b1[l]),
                          upd_w2[l], row(upd_b2[l]),
                          out_w1, row(out_b1), ow2p, ob2p)
    return pred[:, :out_dim]


# speedup vs baseline: 1.1874x; 1.1874x over previous
import functools

import jax
import jax.numpy as jnp
from jax import lax
from jax.experimental import pallas as pl
from jax.experimental.pallas import tpu as pltpu
from jax.experimental.pallas import tpu_sc as plsc

_N = 10000
_E = 320000
_DI = 128
_DE = 16
_H = 64
_L = 3

_NC = 2
_NS = 16
_NW = _NC * _NS
_EW = _E // _NW
_C = 80
_CH = _EW // _C
_NP = 10240
_RPT = _NP // _NS
_NB = 5

_BN = 2000
_BE = 4000


def _mm(a, b):
    return lax.dot_general(a, b, (((1,), (0,)), ((), ())),
                           preferred_element_type=jnp.float32)


def _sc_mesh():
    return plsc.VectorSubcoreMesh(core_axis_name="c", subcore_axis_name="s",
                                  num_cores=_NC, num_subcores=_NS)


def _replicated(shape):
    return pl.BlockSpec(shape, lambda i: (0,) * len(shape))


def _gather_rows(tbl, idx4):
    @functools.partial(
        pl.kernel,
        out_type=jax.ShapeDtypeStruct((_E, _H), jnp.float32),
        mesh=_sc_mesh(),
        compiler_params=pltpu.CompilerParams(use_tc_tiling_on_sc=False),
        scratch_types=[
            pltpu.VMEM((_CH, _C), jnp.int32),
            [pltpu.VMEM((_C, _H), jnp.float32)] * _NB,
            [pltpu.SemaphoreType.DMA] * _NB,
            [pltpu.SemaphoreType.DMA] * _NB,
        ],
    )
    def k(tbl_hbm, idx_hbm, out_hbm, idx_v, bufs, gsems, wsems):
        c = lax.axis_index("c")
        s = lax.axis_index("s")
        base = (c * _NS + s) * _EW
        pltpu.sync_copy(idx_hbm.at[c, s], idx_v)
        pltpu.async_copy(tbl_hbm.at[idx_v.at[0]], bufs[0], gsems[0])
        pltpu.async_copy(tbl_hbm.at[idx_v.at[1]], bufs[1], gsems[1])

        def out_at(j):
            return out_hbm.at[pl.ds(base + j * _C, _C)]

        def body(i, carry):
            for b in range(_NB):
                j = i * _NB + b
                b2 = (b + 2) % _NB

                @pl.when(j >= 3)
                def _():
                    pltpu.make_async_copy(bufs[b2], out_at(0), wsems[b2]).wait()

                @pl.when(j + 2 < _CH)
                def _():
                    pltpu.async_copy(tbl_hbm.at[idx_v.at[j + 2]], bufs[b2],
                                     gsems[b2])

                pltpu.make_async_copy(tbl_hbm.at[idx_v.at[j]], bufs[b],
                                      gsems[b]).wait()
                pltpu.async_copy(bufs[b], out_at(j), wsems[b])
            return carry

        lax.fori_loop(0, _CH // _NB, body, 0)
        for j in range(_CH - 3, _CH):
            b = j % _NB
            pltpu.make_async_copy(bufs[b], out_at(0), wsems[b]).wait()

    return k(tbl, idx4)


def _scatter_add(msgs, idx4, z_h):
    @functools.partial(
        pl.kernel,
        out_type=jax.ShapeDtypeStruct((_NC, _NP, _H), jnp.float32),
        mesh=_sc_mesh(),
        compiler_params=pltpu.CompilerParams(use_tc_tiling_on_sc=False),
        scratch_types=[
            pltpu.VMEM((_CH, _C), jnp.int32),
            [pltpu.VMEM((_C, _H), jnp.float32)] * _NB,
            [pltpu.SemaphoreType.DMA] * _NB,
            [pltpu.SemaphoreType.DMA] * _NB,
            pltpu.VMEM_SHARED((_NP, _H), jnp.float32),
        ],
    )
    def k(msg_hbm, idx_hbm, z_hbm, sum_hbm, idx_v, bufs, lsems, asems, acc):
        c = lax.axis_index("c")
        s = lax.axis_index("s")
        base = (c * _NS + s) * _EW
        rows = pl.ds(s * _RPT, _RPT)
        pltpu.sync_copy(idx_hbm.at[c, s], idx_v)
        pltpu.sync_copy(z_hbm.at[rows], acc.at[rows])
        plsc.subcore_barrier()

        def msg_at(j):
            return msg_hbm.at[pl.ds(base + j * _C, _C)]

        pltpu.async_copy(msg_at(0), bufs[0], lsems[0])
        pltpu.async_copy(msg_at(1), bufs[1], lsems[1])

        def body(i, carry):
            for b in range(_NB):
                j = i * _NB + b
                b2 = (b + 2) % _NB

                @pl.when(j >= 3)
                def _():
                    pltpu.make_async_copy(bufs[b2], acc.at[idx_v.at[0]],
                                          asems[b2]).wait()

                @pl.when(j + 2 < _CH)
                def _():
                    pltpu.async_copy(msg_at(j + 2), bufs[b2], lsems[b2])

                pltpu.make_async_copy(msg_at(j), bufs[b], lsems[b]).wait()
                pltpu.async_copy(bufs[b], acc.at[idx_v.at[j]], asems[b],
                                 add=True)
            return carry

        lax.fori_loop(0, _CH // _NB, body, 0)
        for j in range(_CH - 3, _CH):
            b = j % _NB
            pltpu.make_async_copy(bufs[b], acc.at[idx_v.at[0]], asems[b]).wait()
        plsc.subcore_barrier()
        pltpu.sync_copy(acc.at[rows], sum_hbm.at[c, rows])

    return k(msgs, idx4, z_h)


def _scatter_add_cnt(msgs, idx4, z_h, z_c, o_c):
    @functools.partial(
        pl.kernel,
        out_type=[jax.ShapeDtypeStruct((_NC, _NP, _H), jnp.float32),
                  jax.ShapeDtypeStruct((_NC, _NP, 16), jnp.float32)],
        mesh=_sc_mesh(),
        compiler_params=pltpu.CompilerParams(use_tc_tiling_on_sc=False),
        scratch_types=[
            pltpu.VMEM((_CH, _C), jnp.int32),
            [pltpu.VMEM((_C, _H), jnp.float32)] * _NB,
            [pltpu.SemaphoreType.DMA] * _NB,
            [pltpu.SemaphoreType.DMA] * _NB,
            pltpu.VMEM((_C, 16), jnp.float32),
            pltpu.SemaphoreType.DMA,
            pltpu.VMEM_SHARED((_NP, _H), jnp.float32),
            pltpu.VMEM_SHARED((_NP, 16), jnp.float32),
        ],
    )
    def k(msg_hbm, idx_hbm, z_hbm, zc_hbm, on_hbm, sum_hbm, cnt_hbm,
          idx_v, bufs, lsems, asems, one_v, csem, acc, cacc):
        c = lax.axis_index("c")
        s = lax.axis_index("s")
        base = (c * _NS + s) * _EW
        rows = pl.ds(s * _RPT, _RPT)
        pltpu.sync_copy(idx_hbm.at[c, s], idx_v)
        pltpu.sync_copy(on_hbm, one_v)
        pltpu.sync_copy(z_hbm.at[rows], acc.at[rows])
        pltpu.sync_copy(zc_hbm.at[rows], cacc.at[rows])
        plsc.subcore_barrier()

        def msg_at(j):
            return msg_hbm.at[pl.ds(base + j * _C, _C)]

        pltpu.async_copy(msg_at(0), bufs[0], lsems[0])
        pltpu.async_copy(msg_at(1), bufs[1], lsems[1])

        def body(i, carry):
            for b in range(_NB):
                j = i * _NB + b
                b2 = (b + 2) % _NB

                @pl.when(j >= 3)
                def _():
                    pltpu.make_async_copy(bufs[b2], acc.at[idx_v.at[0]],
                                          asems[b2]).wait()

                @pl.when(j + 2 < _CH)
                def _():
                    pltpu.async_copy(msg_at(j + 2), bufs[b2], lsems[b2])

                @pl.when(j >= 1)
                def _():
                    pltpu.make_async_copy(one_v, cacc.at[idx_v.at[0]],
                                          csem).wait()

                pltpu.make_async_copy(msg_at(j), bufs[b], lsems[b]).wait()
                pltpu.async_copy(bufs[b], acc.at[idx_v.at[j]], asems[b],
                                 add=True)
                pltpu.async_copy(one_v, cacc.at[idx_v.at[j]], csem, add=True)
            return carry

        lax.fori_loop(0, _CH // _NB, body, 0)
        for j in range(_CH - 3, _CH):
            b = j % _NB
            pltpu.make_async_copy(bufs[b], acc.at[idx_v.at[0]], asems[b]).wait()
        pltpu.make_async_copy(one_v, cacc.at[idx_v.at[0]], csem).wait()
        plsc.subcore_barrier()
        pltpu.sync_copy(acc.at[rows], sum_hbm.at[c, rows])
        pltpu.sync_copy(cacc.at[rows], cnt_hbm.at[c, rows])

    return k(msgs, idx4, z_h, z_c, o_c)


def _encode(x, w1, b1, w2, b2, mt, mb):
    def body(x_r, w1_r, b1_r, w2_r, b2_r, mt_r, mb_r, h_r, t_r):
        h = jnp.maximum(_mm(x_r[...], w1_r[...]) + b1_r[...], 0.0)
        h = jnp.maximum(_mm(h, w2_r[...]) + b2_r[...], 0.0)
        h_r[...] = h
        t_r[...] = _mm(h, mt_r[...]) + mb_r[...]

    return pl.pallas_call(
        body,
        grid=(_N // _BN,),
        in_specs=[
            pl.BlockSpec((_BN, _DI), lambda i: (i, 0)),
            _replicated((_DI, _H)), _replicated((1, _H)),
            _replicated((_H, _H)), _replicated((1, _H)),
            _replicated((_H, _H)), _replicated((1, _H)),
        ],
        out_specs=[pl.BlockSpec((_BN, _H), lambda i: (i, 0))] * 2,
        out_shape=[jax.ShapeDtypeStruct((_N, _H), jnp.float32)] * 2,
    )(x, w1, b1, w2, b2, mt, mb)


def _messages(g, ef, eew, eeb, w1b, w2, b2):
    def body(g_r, ef_r, eew_r, eeb_r, w1b_r, w2_r, b2_r, o_r):
        eh = jnp.maximum(_mm(ef_r[...], eew_r[...]) + eeb_r[...], 0.0)
        hid = jnp.maximum(g_r[...] + _mm(eh, w1b_r[...]), 0.0)
        o_r[...] = jnp.maximum(_mm(hid, w2_r[...]) + b2_r[...], 0.0)

    return pl.pallas_call(
        body,
        grid=(_E // _BE,),
        in_specs=[
            pl.BlockSpec((_BE, _H), lambda i: (i, 0)),
            pl.BlockSpec((_BE, _DE), lambda i: (i, 0)),
            _replicated((_DE, _H)), _replicated((1, _H)),
            _replicated((_H, _H)), _replicated((_H, _H)), _replicated((1, _H)),
        ],
        out_specs=pl.BlockSpec((_BE, _H), lambda i: (i, 0)),
        out_shape=jax.ShapeDtypeStruct((_E, _H), jnp.float32),
    )(g, ef, eew, eeb, w1b, w2, b2)


def _update(h, p0, p1, c0, c1, uw1t, uw1b, ub1, uw2, ub2, mt, mb):
    def body(h_r, p0_r, p1_r, c0_r, c1_r, uw1t_r, uw1b_r, ub1_r, uw2_r, ub2_r,
             mt_r, mb_r, hn_r, tn_r):
        cnt = c0_r[...] + c1_r[...]
        agg = (p0_r[...] + p1_r[...]) / jnp.maximum(cnt[:, :1], 1.0)
        hid = jnp.maximum(_mm(h_r[...], uw1t_r[...]) + _mm(agg, uw1b_r[...])
                          + ub1_r[...], 0.0)
        hn = jnp.maximum(_mm(hid, uw2_r[...]) + ub2_r[...], 0.0)
        hn_r[...] = hn
        tn_r[...] = _mm(hn, mt_r[...]) + mb_r[...]

    return pl.pallas_call(
        body,
        grid=(_N // _BN,),
        in_specs=[
            pl.BlockSpec((_BN, _H), lambda i: (i, 0)),
            pl.BlockSpec((_BN, _H), lambda i: (i, 0)),
            pl.BlockSpec((_BN, _H), lambda i: (i, 0)),
            pl.BlockSpec((_BN, 16), lambda i: (i, 0)),
            pl.BlockSpec((_BN, 16), lambda i: (i, 0)),
            _replicated((_H, _H)), _replicated((_H, _H)), _replicated((1, _H)),
            _replicated((_H, _H)), _replicated((1, _H)),
            _replicated((_H, _H)), _replicated((1, _H)),
        ],
        out_specs=[pl.BlockSpec((_BN, _H), lambda i: (i, 0))] * 2,
        out_shape=[jax.ShapeDtypeStruct((_N, _H), jnp.float32)] * 2,
    )(h, p0, p1, c0, c1, uw1t, uw1b, ub1, uw2, ub2, mt, mb)


def _final(h, p0, p1, c0, c1, uw1t, uw1b, ub1, uw2, ub2, ow1, ob1, ow2p, ob2p):
    def body(h_r, p0_r, p1_r, c0_r, c1_r, uw1t_r, uw1b_r, ub1_r, uw2_r, ub2_r,
             ow1_r, ob1_r, ow2_r, ob2_r, o_r):
        cnt = c0_r[...] + c1_r[...]
        agg = (p0_r[...] + p1_r[...]) / jnp.maximum(cnt[:, :1], 1.0)
        hid = jnp.maximum(_mm(h_r[...], uw1t_r[...]) + _mm(agg, uw1b_r[...])
                          + ub1_r[...], 0.0)
        hn = jnp.maximum(_mm(hid, uw2_r[...]) + ub2_r[...], 0.0)
        hid2 = jnp.maximum(_mm(hn, ow1_r[...]) + ob1_r[...], 0.0)
        pred = _mm(hid2, ow2_r[...]) + ob2_r[...]
        o_r[...] = (2.0 * jnp.pi) / (1.0 + jnp.exp(-pred))

    return pl.pallas_call(
        body,
        grid=(_N // _BN,),
        in_specs=[
            pl.BlockSpec((_BN, _H), lambda i: (i, 0)),
            pl.BlockSpec((_BN, _H), lambda i: (i, 0)),
            pl.BlockSpec((_BN, _H), lambda i: (i, 0)),
            pl.BlockSpec((_BN, 16), lambda i: (i, 0)),
            pl.BlockSpec((_BN, 16), lambda i: (i, 0)),
            _replicated((_H, _H)), _replicated((_H, _H)), _replicated((1, _H)),
            _replicated((_H, _H)), _replicated((1, _H)),
            _replicated((_H, _H)), _replicated((1, _H)),
            _replicated((_H, 128)), _replicated((1, 128)),
        ],
        out_specs=pl.BlockSpec((_BN, 128), lambda i: (i, 0)),
        out_shape=jax.ShapeDtypeStruct((_N, 128), jnp.float32),
    )(h, p0, p1, c0, c1, uw1t, uw1b, ub1, uw2, ub2, ow1, ob1, ow2p, ob2p)


def kernel(node_features, edge_index, edge_features, ne_w1, ne_b1, ne_w2, ne_b2,
           ee_w1, ee_b1, msg_w1, msg_b1, msg_w2, msg_b2,
           upd_w1, upd_b1, upd_w2, upd_b2, out_w1, out_b1, out_w2, out_b2):
    row = lambda v: v.reshape(1, -1)
    out_dim = out_w2.shape[1]
    src4 = edge_index[0].reshape(_NC, _NS, _CH, _C)
    dst4 = edge_index[1].reshape(_NC, _NS, _CH, _C)
    z_h = jnp.zeros((_NP, _H), jnp.float32)
    z_c = jnp.zeros((_NP, 16), jnp.float32)
    o_c = jnp.ones((_C, 16), jnp.float32)
    ow2p = jnp.pad(out_w2, ((0, 0), (0, 128 - out_dim)))
    ob2p = row(jnp.pad(out_b2, (0, 128 - out_dim)))

    h, t = _encode(node_features, ne_w1, row(ne_b1), ne_w2, row(ne_b2),
                   msg_w1[0, :_H], row(msg_b1[0]))
    c0 = c1 = None
    for l in range(_L):
        g = _gather_rows(t, src4)
        msgs = _messages(g, edge_features, ee_w1, row(ee_b1),
                         msg_w1[l, _H:], msg_w2[l], row(msg_b2[l]))
        if l == 0:
            sums, cnts = _scatter_add_cnt(msgs, dst4, z_h, z_c, o_c)
            c0, c1 = cnts[0], cnts[1]
        else:
            sums = _scatter_add(msgs, dst4, z_h)
        if l < _L - 1:
            h, t = _update(h, sums[0], sums[1], c0, c1,
                           upd_w1[l, :_H], upd_w1[l, _H:], row(upd_b1[l]),
                           upd_w2[l], row(upd_b2[l]),
                           msg_w1[l + 1, :_H], row(msg_b1[l + 1]))
        else:
            pred = _final(h, sums[0], sums[1], c0, c1,
                          upd_w1[l, :_H], upd_w1[l, _H:], row(upd_b1[l]),
                          upd_w2[l], row(upd_b2[l]),
                          out_w1, row(out_b1), ow2p, ob2p)
    return pred[:, :out_dim]
